# Initial kernel scaffold; baseline (speedup 1.0000x reference)
import jax
import jax.numpy as jnp
from jax.experimental import pallas as pl


def kernel(x, codebook):
    raise NotImplementedError("write your pallas kernel here")


# speedup vs baseline: 1.7897x; 1.7897x over previous
import functools

import jax
import jax.numpy as jnp
from jax import lax
from jax.experimental import pallas as pl
from jax.experimental.pallas import tpu as pltpu
from jax.experimental.pallas import tpu_sc as plsc

B, T, D, K = 8, 1024, 32, 8192
N_TOK = B * T
TOK_TILE = 256
N_TILES = N_TOK // TOK_TILE

_NC, _NS = 2, 16
_NW = _NC * _NS
_TOK_PER_W = N_TOK // _NW


def _argmax_body(x_ref, cb_ref, ids_ref):
    logits = lax.dot_general(
        x_ref[...], cb_ref[...],
        dimension_numbers=(((1,), (1,)), ((), ())),
        preferred_element_type=jnp.float32,
    )
    ids = jnp.argmax(logits, axis=1).astype(jnp.int32)
    ids_ref[0, 0, :] = ids


def _tc_argmax(x2d, codebook):
    ids3 = pl.pallas_call(
        _argmax_body,
        grid=(N_TILES,),
        in_specs=[
            pl.BlockSpec((TOK_TILE, D), lambda i: (i, 0)),
            pl.BlockSpec((K, D), lambda i: (0, 0)),
        ],
        out_specs=pl.BlockSpec((1, 1, TOK_TILE), lambda i: (i, 0, 0)),
        out_shape=jax.ShapeDtypeStruct((N_TILES, 1, TOK_TILE), jnp.int32),
    )(x2d, codebook)
    return ids3.reshape(N_TOK)


@functools.cache
def _make_sc_gather():
    mesh = plsc.VectorSubcoreMesh(core_axis_name="c", subcore_axis_name="s")

    @functools.partial(
        pl.kernel,
        mesh=mesh,
        out_type=jax.ShapeDtypeStruct((N_TOK, D), jnp.float32),
        scratch_types=[
            pltpu.VMEM((_TOK_PER_W,), jnp.int32),
            pltpu.VMEM((_TOK_PER_W, D), jnp.float32),
            pltpu.SemaphoreType.DMA,
        ],
        compiler_params=pltpu.CompilerParams(use_tc_tiling_on_sc=False),
    )
    def _sc_gather(ids_hbm, table_hbm, out_hbm, idx_v, rows_v, sem):
        wid = lax.axis_index("s") * _NC + lax.axis_index("c")
        base = wid * _TOK_PER_W
        pltpu.sync_copy(ids_hbm.at[pl.ds(base, _TOK_PER_W)], idx_v)
        pltpu.async_copy(table_hbm.at[idx_v], rows_v, sem).wait()
        pltpu.sync_copy(rows_v, out_hbm.at[pl.ds(base, _TOK_PER_W)])

    return _sc_gather


def kernel(x, codebook):
    x2d = x.reshape(N_TOK, D)
    ids = _tc_argmax(x2d, codebook)
    out = _make_sc_gather()(ids, codebook)
    return out.reshape(B, T, D)


# speedup vs baseline: 1.9092x; 1.0668x over previous
import functools

import jax
import jax.numpy as jnp
from jax import lax
from jax.experimental import pallas as pl
from jax.experimental.pallas import tpu as pltpu
from jax.experimental.pallas import tpu_sc as plsc

B, T, D, K = 8, 1024, 32, 8192
N_TOK = B * T
TOK_TILE = 512
N_TILES = N_TOK // TOK_TILE

_NC, _NS = 2, 16
_NW = _NC * _NS
_TOK_PER_W = N_TOK // _NW


def _argmax_body(x_ref, cb_ref, ids_ref):
    logits = lax.dot_general(
        x_ref[...], cb_ref[...],
        dimension_numbers=(((1,), (1,)), ((), ())),
        preferred_element_type=jnp.float32,
    )
    ids = jnp.argmax(logits, axis=1).astype(jnp.int32)
    ids_ref[0, 0, :] = ids


def _tc_argmax(x2d, codebook):
    ids3 = pl.pallas_call(
        _argmax_body,
        grid=(N_TILES,),
        in_specs=[
            pl.BlockSpec((TOK_TILE, D), lambda i: (i, 0)),
            pl.BlockSpec((K, D), lambda i: (0, 0)),
        ],
        out_specs=pl.BlockSpec((1, 1, TOK_TILE), lambda i: (i, 0, 0)),
        out_shape=jax.ShapeDtypeStruct((N_TILES, 1, TOK_TILE), jnp.int32),
    )(x2d, codebook)
    return ids3.reshape(N_TOK)


@functools.cache
def _make_sc_gather():
    mesh = plsc.VectorSubcoreMesh(core_axis_name="c", subcore_axis_name="s")

    @functools.partial(
        pl.kernel,
        mesh=mesh,
        out_type=jax.ShapeDtypeStruct((N_TOK, D), jnp.float32),
        scratch_types=[
            pltpu.VMEM((_TOK_PER_W,), jnp.int32),
            pltpu.VMEM((_TOK_PER_W, D), jnp.float32),
            pltpu.SemaphoreType.DMA,
        ],
        compiler_params=pltpu.CompilerParams(use_tc_tiling_on_sc=False),
    )
    def _sc_gather(ids_hbm, table_hbm, out_hbm, idx_v, rows_v, sem):
        wid = lax.axis_index("s") * _NC + lax.axis_index("c")
        base = wid * _TOK_PER_W
        pltpu.sync_copy(ids_hbm.at[pl.ds(base, _TOK_PER_W)], idx_v)
        pltpu.async_copy(table_hbm.at[idx_v], rows_v, sem).wait()
        pltpu.sync_copy(rows_v, out_hbm.at[pl.ds(base, _TOK_PER_W)])

    return _sc_gather


def kernel(x, codebook):
    x2d = x.reshape(N_TOK, D)
    ids = _tc_argmax(x2d, codebook)
    out = _make_sc_gather()(ids, codebook)
    return out.reshape(B, T, D)


# speedup vs baseline: 1.9170x; 1.0041x over previous
import functools

import jax
import jax.numpy as jnp
from jax import lax
from jax.experimental import pallas as pl
from jax.experimental.pallas import tpu as pltpu
from jax.experimental.pallas import tpu_sc as plsc

B, T, D, K = 8, 1024, 32, 8192
N_TOK = B * T
TOK_TILE = 1024
N_TILES = N_TOK // TOK_TILE
LW = 128

_NC, _NS = 2, 16
_NW = _NC * _NS
_TOK_PER_W = N_TOK // _NW


def _argmax_body(x_ref, cb_ref, ids_ref, tab_ref):
    logits = lax.dot_general(
        x_ref[0], cb_ref[...],
        dimension_numbers=(((1,), (1,)), ((), ())),
        preferred_element_type=jnp.float32,
    )
    ids = jnp.argmax(logits, axis=1).astype(jnp.int32)
    ids_ref[...] = ids.reshape(TOK_TILE // LW, LW)

    @pl.when(pl.program_id(0) == 0)
    def _():
        tab_ref[...] = jnp.pad(cb_ref[...], ((0, 0), (0, LW - D)))


def _tc_argmax(x, codebook):
    rows_per_tile = TOK_TILE // LW
    return pl.pallas_call(
        _argmax_body,
        grid=(N_TILES,),
        in_specs=[
            pl.BlockSpec((1, TOK_TILE, D), lambda i: (i, 0, 0)),
            pl.BlockSpec((K, D), lambda i: (0, 0)),
        ],
        out_specs=[
            pl.BlockSpec((rows_per_tile, LW), lambda i: (i, 0)),
            pl.BlockSpec((K, LW), lambda i: (0, 0)),
        ],
        out_shape=[
            jax.ShapeDtypeStruct((N_TOK // LW, LW), jnp.int32),
            jax.ShapeDtypeStruct((K, LW), jnp.float32),
        ],
    )(x, codebook)


@functools.cache
def _make_sc_gather():
    mesh = plsc.VectorSubcoreMesh(core_axis_name="c", subcore_axis_name="s")
    rows_per_w = _TOK_PER_W // LW

    @functools.partial(
        pl.kernel,
        mesh=mesh,
        out_type=jax.ShapeDtypeStruct((N_TOK, LW), jnp.float32),
        scratch_types=[
            pltpu.VMEM((rows_per_w, LW), jnp.int32),
            pltpu.VMEM((LW, LW), jnp.float32),
            pltpu.SemaphoreType.DMA,
        ],
        compiler_params=pltpu.CompilerParams(use_tc_tiling_on_sc=True),
    )
    def _sc_gather(ids_hbm, table_hbm, out_hbm, idx_v, rows_v, sem):
        wid = lax.axis_index("s") * _NC + lax.axis_index("c")
        pltpu.sync_copy(ids_hbm.at[pl.ds(wid * rows_per_w, rows_per_w)], idx_v)
        for h in range(rows_per_w):
            pltpu.async_copy(table_hbm.at[idx_v.at[h]], rows_v, sem).wait()
            base = wid * _TOK_PER_W + h * LW
            pltpu.sync_copy(rows_v, out_hbm.at[pl.ds(base, LW)])

    return _sc_gather


def kernel(x, codebook):
    ids, table = _tc_argmax(x, codebook)
    out = _make_sc_gather()(ids, table)
    return out[:, :D].reshape(B, T, D)
